# baseline (speedup 1.0000x reference)
import functools

import jax
import jax.numpy as jnp
from jax.experimental import pallas as pl
from jax.experimental.pallas import tpu as pltpu

CODEBOOK_SIZE = 1024
LATENT_DIM = 64
RQ_LEVELS = 4
BLOCK_N = 512


def _level_kernel(last, r_ref, rsq_ref, qsum_ref, z_ref, cb_ref, cbsq_ref,
                  rout_ref, qsout_ref, idx_ref, usage_ref, loss_ref):
    step = pl.program_id(0)

    @pl.when(step == 0)
    def _init():
        usage_ref[...] = jnp.zeros_like(usage_ref)
        loss_ref[...] = jnp.zeros_like(loss_ref)

    r = r_ref[...]
    cb = cb_ref[...]
    cross = jax.lax.dot_general(
        r, cb, (((1,), (1,)), ((), ())),
        preferred_element_type=jnp.float32)
    d = rsq_ref[...] - 2.0 * cross + cbsq_ref[...]
    dmin = jnp.min(d, axis=1, keepdims=True)
    col_iota = jax.lax.broadcasted_iota(
        jnp.int32, (BLOCK_N, CODEBOOK_SIZE), 1)
    idx = jnp.min(jnp.where(d == dmin, col_iota, CODEBOOK_SIZE),
                  axis=1, keepdims=True)
    onehot = (col_iota == idx).astype(jnp.float32)
    q = jax.lax.dot_general(
        onehot, cb, (((1,), (0,)), ((), ())),
        precision=jax.lax.Precision.HIGHEST,
        preferred_element_type=jnp.float32)

    diff = r - q
    rout_ref[...] = diff
    qs = qsum_ref[...] + q
    if last:
        z = z_ref[...]
        qsout_ref[...] = z + (qs - z)
    else:
        qsout_ref[...] = qs
    idx_ref[...] = idx
    usage_ref[...] += jnp.sum(onehot, axis=0, keepdims=True)
    loss_ref[...] += jnp.reshape(jnp.sum(diff * diff), (1, 1))


def _make_level(n, last):
    row = lambda i: (i, 0)
    rep = lambda i: (0, 0)
    return pl.pallas_call(
        functools.partial(_level_kernel, last),
        grid=(n // BLOCK_N,),
        in_specs=[
            pl.BlockSpec((BLOCK_N, LATENT_DIM), row),
            pl.BlockSpec((BLOCK_N, 1), row),
            pl.BlockSpec((BLOCK_N, LATENT_DIM), row),
            pl.BlockSpec((BLOCK_N, LATENT_DIM), row),
            pl.BlockSpec((CODEBOOK_SIZE, LATENT_DIM), rep),
            pl.BlockSpec((1, CODEBOOK_SIZE), rep),
        ],
        out_specs=[
            pl.BlockSpec((BLOCK_N, LATENT_DIM), row),
            pl.BlockSpec((BLOCK_N, LATENT_DIM), row),
            pl.BlockSpec((BLOCK_N, 1), row),
            pl.BlockSpec((1, CODEBOOK_SIZE), rep),
            pl.BlockSpec((1, 1), rep),
        ],
        out_shape=[
            jax.ShapeDtypeStruct((n, LATENT_DIM), jnp.float32),
            jax.ShapeDtypeStruct((n, LATENT_DIM), jnp.float32),
            jax.ShapeDtypeStruct((n, 1), jnp.int32),
            jax.ShapeDtypeStruct((1, CODEBOOK_SIZE), jnp.float32),
            jax.ShapeDtypeStruct((1, 1), jnp.float32),
        ],
        compiler_params=pltpu.CompilerParams(
            dimension_semantics=("arbitrary",)),
    )


@jax.jit
def kernel(z, codebook):
    n = z.shape[0]
    cbsq = (codebook ** 2).sum(axis=1)[None, :]
    inv_count = 1.0 / (n * LATENT_DIM)

    residual = z
    qsum = jnp.zeros_like(z)
    loss = jnp.float32(0.0)
    usage = jnp.zeros((1, CODEBOOK_SIZE), jnp.float32)
    codes = []
    for lvl in range(RQ_LEVELS):
        rsq = (residual ** 2).sum(axis=1, keepdims=True)
        residual, qsum, idx, u, s = _make_level(n, lvl == RQ_LEVELS - 1)(
            residual, rsq, qsum, z, codebook, cbsq)
        codes.append(idx[:, 0])
        usage = usage + u
        loss = loss + s[0, 0] * inv_count

    return qsum, loss, jnp.stack(codes, axis=1), usage[0]


# speedup vs baseline: 1.2590x; 1.2590x over previous
import functools

import jax
import jax.numpy as jnp
from jax.experimental import pallas as pl
from jax.experimental.pallas import tpu as pltpu

CODEBOOK_SIZE = 1024
LATENT_DIM = 64
RQ_LEVELS = 4
BLOCK_N = 512


def _level_kernel(last, r_ref, rsq_ref, qsum_ref, z_ref, cb_ref, cbsq_ref,
                  cbhi_ref, cbmid_ref, cblo_ref,
                  rout_ref, qsout_ref, idx_ref, usage_ref, loss_ref):
    step = pl.program_id(0)

    @pl.when(step == 0)
    def _init():
        usage_ref[...] = jnp.zeros_like(usage_ref)
        loss_ref[...] = jnp.zeros_like(loss_ref)

    r = r_ref[...]
    cb = cb_ref[...]
    cross = jax.lax.dot_general(
        r, cb, (((1,), (1,)), ((), ())),
        preferred_element_type=jnp.float32)
    d = rsq_ref[...] - 2.0 * cross + cbsq_ref[...]
    dmin = jnp.min(d, axis=1, keepdims=True)
    col_iota = jax.lax.broadcasted_iota(
        jnp.int32, (BLOCK_N, CODEBOOK_SIZE), 1)
    idx = jnp.min(jnp.where(d == dmin, col_iota, CODEBOOK_SIZE),
                  axis=1, keepdims=True)
    onehot = (col_iota == idx).astype(jnp.float32)
    gdot = lambda ref: jax.lax.dot_general(
        onehot, ref[...], (((1,), (0,)), ((), ())),
        preferred_element_type=jnp.float32)
    q = (gdot(cbhi_ref) + gdot(cbmid_ref)) + gdot(cblo_ref)

    diff = r - q
    rout_ref[...] = diff
    qs = qsum_ref[...] + q
    if last:
        z = z_ref[...]
        qsout_ref[...] = z + (qs - z)
    else:
        qsout_ref[...] = qs
    idx_ref[...] = idx
    usage_ref[...] += jnp.sum(onehot, axis=0, keepdims=True)
    loss_ref[...] += jnp.reshape(jnp.sum(diff * diff), (1, 1))


def _make_level(n, last):
    row = lambda i: (i, 0)
    rep = lambda i: (0, 0)
    return pl.pallas_call(
        functools.partial(_level_kernel, last),
        grid=(n // BLOCK_N,),
        in_specs=[
            pl.BlockSpec((BLOCK_N, LATENT_DIM), row),
            pl.BlockSpec((BLOCK_N, 1), row),
            pl.BlockSpec((BLOCK_N, LATENT_DIM), row),
            pl.BlockSpec((BLOCK_N, LATENT_DIM), row),
            pl.BlockSpec((CODEBOOK_SIZE, LATENT_DIM), rep),
            pl.BlockSpec((1, CODEBOOK_SIZE), rep),
            pl.BlockSpec((CODEBOOK_SIZE, LATENT_DIM), rep),
            pl.BlockSpec((CODEBOOK_SIZE, LATENT_DIM), rep),
            pl.BlockSpec((CODEBOOK_SIZE, LATENT_DIM), rep),
        ],
        out_specs=[
            pl.BlockSpec((BLOCK_N, LATENT_DIM), row),
            pl.BlockSpec((BLOCK_N, LATENT_DIM), row),
            pl.BlockSpec((BLOCK_N, 1), row),
            pl.BlockSpec((1, CODEBOOK_SIZE), rep),
            pl.BlockSpec((1, 1), rep),
        ],
        out_shape=[
            jax.ShapeDtypeStruct((n, LATENT_DIM), jnp.float32),
            jax.ShapeDtypeStruct((n, LATENT_DIM), jnp.float32),
            jax.ShapeDtypeStruct((n, 1), jnp.int32),
            jax.ShapeDtypeStruct((1, CODEBOOK_SIZE), jnp.float32),
            jax.ShapeDtypeStruct((1, 1), jnp.float32),
        ],
        compiler_params=pltpu.CompilerParams(
            dimension_semantics=("arbitrary",)),
    )


@jax.jit
def kernel(z, codebook):
    n = z.shape[0]
    cbsq = (codebook ** 2).sum(axis=1)[None, :]
    def trunc16(x):
        u = jax.lax.bitcast_convert_type(x, jnp.uint32)
        return jax.lax.bitcast_convert_type(
            u & jnp.uint32(0xFFFF0000), jnp.float32)
    cb_hi = trunc16(codebook)
    rem = codebook - cb_hi
    cb_mid = trunc16(rem)
    cb_lo = rem - cb_mid
    inv_count = 1.0 / (n * LATENT_DIM)

    residual = z
    qsum = jnp.zeros_like(z)
    loss = jnp.float32(0.0)
    usage = jnp.zeros((1, CODEBOOK_SIZE), jnp.float32)
    codes = []
    for lvl in range(RQ_LEVELS):
        rsq = (residual ** 2).sum(axis=1, keepdims=True)
        residual, qsum, idx, u, s = _make_level(n, lvl == RQ_LEVELS - 1)(
            residual, rsq, qsum, z, codebook, cbsq, cb_hi, cb_mid, cb_lo)
        codes.append(idx[:, 0])
        usage = usage + u
        loss = loss + s[0, 0] * inv_count

    return qsum, loss, jnp.stack(codes, axis=1), usage[0]


# speedup vs baseline: 1.3511x; 1.0732x over previous
import functools

import jax
import jax.numpy as jnp
from jax.experimental import pallas as pl
from jax.experimental.pallas import tpu as pltpu

CODEBOOK_SIZE = 1024
LATENT_DIM = 64
RQ_LEVELS = 4
BLOCK_N = 1024


def _level_kernel(last, r_ref, rsq_ref, qsum_ref, z_ref, cb_ref, cbsq_ref,
                  cbhi_ref, cbmid_ref, cblo_ref,
                  rout_ref, qsout_ref, idx_ref, usage_ref, loss_ref):
    step = pl.program_id(0)

    @pl.when(step == 0)
    def _init():
        usage_ref[...] = jnp.zeros_like(usage_ref)
        loss_ref[...] = jnp.zeros_like(loss_ref)

    r = r_ref[...]
    cb = cb_ref[...]
    cross = jax.lax.dot_general(
        r, cb, (((1,), (1,)), ((), ())),
        preferred_element_type=jnp.float32)
    d = rsq_ref[...] - 2.0 * cross + cbsq_ref[...]
    dmin = jnp.min(d, axis=1, keepdims=True)
    col_iota = jax.lax.broadcasted_iota(
        jnp.int32, (BLOCK_N, CODEBOOK_SIZE), 1)
    idx = jnp.min(jnp.where(d == dmin, col_iota, CODEBOOK_SIZE),
                  axis=1, keepdims=True)
    onehot = (col_iota == idx).astype(jnp.float32)
    gdot = lambda ref: jax.lax.dot_general(
        onehot, ref[...], (((1,), (0,)), ((), ())),
        preferred_element_type=jnp.float32)
    q = (gdot(cbhi_ref) + gdot(cbmid_ref)) + gdot(cblo_ref)

    diff = r - q
    rout_ref[...] = diff
    qs = qsum_ref[...] + q
    if last:
        z = z_ref[...]
        qsout_ref[...] = z + (qs - z)
    else:
        qsout_ref[...] = qs
    idx_ref[...] = idx
    usage_ref[...] += jnp.sum(onehot, axis=0, keepdims=True)
    loss_ref[...] += jnp.reshape(jnp.sum(diff * diff), (1, 1))


def _make_level(n, last):
    row = lambda i: (i, 0)
    rep = lambda i: (0, 0)
    return pl.pallas_call(
        functools.partial(_level_kernel, last),
        grid=(n // BLOCK_N,),
        in_specs=[
            pl.BlockSpec((BLOCK_N, LATENT_DIM), row),
            pl.BlockSpec((BLOCK_N, 1), row),
            pl.BlockSpec((BLOCK_N, LATENT_DIM), row),
            pl.BlockSpec((BLOCK_N, LATENT_DIM), row),
            pl.BlockSpec((CODEBOOK_SIZE, LATENT_DIM), rep),
            pl.BlockSpec((1, CODEBOOK_SIZE), rep),
            pl.BlockSpec((CODEBOOK_SIZE, LATENT_DIM), rep),
            pl.BlockSpec((CODEBOOK_SIZE, LATENT_DIM), rep),
            pl.BlockSpec((CODEBOOK_SIZE, LATENT_DIM), rep),
        ],
        out_specs=[
            pl.BlockSpec((BLOCK_N, LATENT_DIM), row),
            pl.BlockSpec((BLOCK_N, LATENT_DIM), row),
            pl.BlockSpec((BLOCK_N, 1), row),
            pl.BlockSpec((1, CODEBOOK_SIZE), rep),
            pl.BlockSpec((1, 1), rep),
        ],
        out_shape=[
            jax.ShapeDtypeStruct((n, LATENT_DIM), jnp.float32),
            jax.ShapeDtypeStruct((n, LATENT_DIM), jnp.float32),
            jax.ShapeDtypeStruct((n, 1), jnp.int32),
            jax.ShapeDtypeStruct((1, CODEBOOK_SIZE), jnp.float32),
            jax.ShapeDtypeStruct((1, 1), jnp.float32),
        ],
        compiler_params=pltpu.CompilerParams(
            dimension_semantics=("arbitrary",)),
    )


@jax.jit
def kernel(z, codebook):
    n = z.shape[0]
    cbsq = (codebook ** 2).sum(axis=1)[None, :]
    def trunc16(x):
        u = jax.lax.bitcast_convert_type(x, jnp.uint32)
        return jax.lax.bitcast_convert_type(
            u & jnp.uint32(0xFFFF0000), jnp.float32)
    cb_hi = trunc16(codebook)
    rem = codebook - cb_hi
    cb_mid = trunc16(rem)
    cb_lo = rem - cb_mid
    inv_count = 1.0 / (n * LATENT_DIM)

    residual = z
    qsum = jnp.zeros_like(z)
    loss = jnp.float32(0.0)
    usage = jnp.zeros((1, CODEBOOK_SIZE), jnp.float32)
    codes = []
    for lvl in range(RQ_LEVELS):
        rsq = (residual ** 2).sum(axis=1, keepdims=True)
        residual, qsum, idx, u, s = _make_level(n, lvl == RQ_LEVELS - 1)(
            residual, rsq, qsum, z, codebook, cbsq, cb_hi, cb_mid, cb_lo)
        codes.append(idx[:, 0])
        usage = usage + u
        loss = loss + s[0, 0] * inv_count

    return qsum, loss, jnp.stack(codes, axis=1), usage[0]
